# baseline (speedup 1.0000x reference)
import functools

import jax
import jax.numpy as jnp
from jax import lax
from jax.experimental import pallas as pl
from jax.experimental.pallas import tpu as pltpu
from jax.experimental.pallas import tpu_sc as plsc

NC = 2
NS = 16
NW = NC * NS
CHUNK = 128


def _sc_mesh():
    return plsc.VectorSubcoreMesh(
        core_axis_name="c", subcore_axis_name="s", num_cores=NC, num_subcores=NS
    )


def _make_gather(ne_pad, n, d, nsg):
    sg = 8 * CHUNK

    @functools.partial(
        pl.kernel,
        mesh=_sc_mesh(),
        out_type=[
            jax.ShapeDtypeStruct((ne_pad, d), jnp.float32),
            jax.ShapeDtypeStruct((ne_pad, d), jnp.float32),
        ],
        scratch_types=[
            pltpu.VMEM((8, CHUNK), jnp.int32),
            pltpu.VMEM((8, CHUNK), jnp.int32),
            pltpu.VMEM((2 * CHUNK, d), jnp.float32),
            pltpu.VMEM((2 * CHUNK, d), jnp.float32),
            pltpu.SemaphoreType.DMA,
            pltpu.SemaphoreType.DMA,
            pltpu.SemaphoreType.DMA,
        ],
    )
    def gather_k(x_hbm, src_hbm, dst_hbm, xs_out, xd_out, sidx, didx, srows, drows, s_ix, s_g, s_wb):
        cid = lax.axis_index("c")
        sid = lax.axis_index("s")
        wid = sid * NC + cid

        def body(s, _):
            base = (wid * nsg + s) * sg
            row0 = (wid * nsg + s) * 8
            ci1 = pltpu.async_copy(src_hbm.at[pl.ds(row0, 8)], sidx, s_ix)
            ci2 = pltpu.async_copy(dst_hbm.at[pl.ds(row0, 8)], didx, s_ix)
            ci1.wait()
            ci2.wait()
            for j in range(4):
                cps = []
                for k in range(2):
                    cps.append(pltpu.async_copy(
                        x_hbm.at[sidx.at[2 * j + k]],
                        srows.at[pl.ds(k * CHUNK, CHUNK)], s_g))
                    cps.append(pltpu.async_copy(
                        x_hbm.at[didx.at[2 * j + k]],
                        drows.at[pl.ds(k * CHUNK, CHUNK)], s_g))
                for cp in cps:
                    cp.wait()
                w1 = pltpu.async_copy(srows, xs_out.at[pl.ds(base + j * 2 * CHUNK, 2 * CHUNK)], s_wb)
                w2 = pltpu.async_copy(drows, xd_out.at[pl.ds(base + j * 2 * CHUNK, 2 * CHUNK)], s_wb)
                w1.wait()
                w2.wait()
            return 0

        lax.fori_loop(0, nsg, body, 0)

    return gather_k


def _make_scatter(ne_pad, n_pad, d, nsg):
    rows_per_tile = n_pad // NS
    sg = 8 * CHUNK

    @functools.partial(
        pl.kernel,
        mesh=_sc_mesh(),
        out_type=jax.ShapeDtypeStruct((NC, n_pad, d), jnp.float32),
        scratch_types=[
            pltpu.VMEM_SHARED((n_pad, d), jnp.float32),
            pltpu.VMEM((8, CHUNK), jnp.int32),
            pltpu.VMEM((2 * CHUNK, d), jnp.float32),
            pltpu.SemaphoreType.DMA,
            pltpu.SemaphoreType.DMA,
        ],
    )
    def scatter_k(m_hbm, dst_hbm, zero_hbm, acc_out, acc, didx, mrows, s_ix, s_m):
        cid = lax.axis_index("c")
        sid = lax.axis_index("s")
        wid = sid * NC + cid

        r0 = sid * rows_per_tile
        pltpu.sync_copy(zero_hbm.at[pl.ds(r0, rows_per_tile)], acc.at[pl.ds(r0, rows_per_tile)])
        plsc.subcore_barrier()

        def body(s, _):
            base = (wid * nsg + s) * sg
            ci = pltpu.async_copy(dst_hbm.at[pl.ds((wid * nsg + s) * 8, 8)], didx, s_ix)
            ci.wait()
            for j in range(4):
                cm = pltpu.async_copy(
                    m_hbm.at[pl.ds(base + j * 2 * CHUNK, 2 * CHUNK)], mrows, s_m)
                cm.wait()
                for k in range(2):
                    pltpu.sync_copy(
                        mrows.at[pl.ds(k * CHUNK, CHUNK)],
                        acc.at[didx.at[2 * j + k]], add=True)
            return 0

        lax.fori_loop(0, nsg, body, 0)
        plsc.subcore_barrier()
        pltpu.sync_copy(acc.at[pl.ds(r0, rows_per_tile)], acc_out.at[cid, pl.ds(r0, rows_per_tile)])

    return scatter_k


def _mlp_body(ne, be, xs_ref, xd_ref, e_ref,
              aw1s_ref, aw1d_ref, aw1e_ref, ab1_ref, aw2_ref, ab2_ref, aw3_ref, ab3_ref,
              mw1s_ref, mw1d_ref, mw1e_ref, mb1_ref, mw2_ref, mb2_ref, out_ref):
    xs = xs_ref[...]
    xd = xd_ref[...]
    e = e_ref[...]
    dot = functools.partial(jnp.dot, preferred_element_type=jnp.float32)
    t = dot(xs, aw1s_ref[...]) + dot(xd, aw1d_ref[...]) + dot(e, aw1e_ref[...]) + ab1_ref[...]
    t = jnp.maximum(t, 0.0)
    t = jnp.maximum(dot(t, aw2_ref[...]) + ab2_ref[...], 0.0)
    a = jnp.sum(t * aw3_ref[...], axis=-1, keepdims=True) + ab3_ref[...]
    h = dot(xs, mw1s_ref[...]) + dot(xd, mw1d_ref[...]) + dot(e, mw1e_ref[...]) + mb1_ref[...]
    h = jnp.maximum(h, 0.0)
    m = dot(h, mw2_ref[...]) + mb2_ref[...]
    msg = jax.nn.sigmoid(a) * m
    row = pl.program_id(0) * be + lax.broadcasted_iota(jnp.int32, msg.shape, 0)
    out_ref[...] = jnp.where(row < ne, msg, 0.0)


def _combine_body(x_ref, a_ref, out_ref):
    out_ref[...] = x_ref[...] + a_ref[0] + a_ref[1]


def kernel(X, E, emb_nodes, emb_edges, edge_index,
           att_W1, att_b1, att_W2, att_b2, att_W3, att_b3,
           msg_W1, msg_b1, msg_W2, msg_b2):
    n, d = X.shape
    ne = E.shape[0]
    sg = 8 * CHUNK
    nsg = -(-ne // (NW * sg))
    ne_pad = nsg * NW * sg
    pad = ne_pad - ne

    src = jnp.concatenate([edge_index[0], jnp.zeros((pad,), jnp.int32)])
    dst = jnp.concatenate([edge_index[1], jnp.zeros((pad,), jnp.int32)])
    src2 = src.reshape(ne_pad // CHUNK, CHUNK)
    dst2 = dst.reshape(ne_pad // CHUNK, CHUNK)
    e_pad = jnp.concatenate([E, jnp.zeros((pad, d), jnp.float32)], axis=0)

    xs, xd = _make_gather(ne_pad, n, d, nsg)(X, src2, dst2)

    be = 2048
    grid = ne_pad // be

    def full(shape):
        return pl.BlockSpec(shape, lambda i: tuple(0 for _ in shape))

    m_arr = pl.pallas_call(
        functools.partial(_mlp_body, ne, be),
        grid=(grid,),
        in_specs=[
            pl.BlockSpec((be, d), lambda i: (i, 0)),
            pl.BlockSpec((be, d), lambda i: (i, 0)),
            pl.BlockSpec((be, d), lambda i: (i, 0)),
            full((d, 96)), full((d, 96)), full((d, 96)), full((1, 96)),
            full((96, 48)), full((1, 48)), full((1, 48)), full((1, 1)),
            full((d, 256)), full((d, 256)), full((d, 256)), full((1, 256)),
            full((256, d)), full((1, d)),
        ],
        out_specs=pl.BlockSpec((be, d), lambda i: (i, 0)),
        out_shape=jax.ShapeDtypeStruct((ne_pad, d), jnp.float32),
        compiler_params=pltpu.CompilerParams(
            dimension_semantics=("arbitrary",),
        ),
    )(
        xs, xd, e_pad,
        att_W1[:d], att_W1[d:2 * d], att_W1[2 * d:], att_b1[None, :],
        att_W2, att_b2[None, :], att_W3.T, att_b3[None, :],
        msg_W1[:d], msg_W1[d:2 * d], msg_W1[2 * d:], msg_b1[None, :],
        msg_W2, msg_b2[None, :],
    )

    n_pad = NS * 8 * (-(-n // (NS * 8)))
    zeros_nd = jnp.zeros((n_pad, d), jnp.float32)
    accs = _make_scatter(ne_pad, n_pad, d, nsg)(m_arr, dst2, zeros_nd)

    bn = 2000
    x_out = pl.pallas_call(
        _combine_body,
        grid=(n // bn,),
        in_specs=[
            pl.BlockSpec((bn, d), lambda i: (i, 0)),
            pl.BlockSpec((NC, bn, d), lambda i: (0, i, 0)),
        ],
        out_specs=pl.BlockSpec((bn, d), lambda i: (i, 0)),
        out_shape=jax.ShapeDtypeStruct((n, d), jnp.float32),
    )(X, accs)

    return (x_out, E)


# speedup vs baseline: 1.0853x; 1.0853x over previous
import functools

import jax
import jax.numpy as jnp
from jax import lax
from jax.experimental import pallas as pl
from jax.experimental.pallas import tpu as pltpu
from jax.experimental.pallas import tpu_sc as plsc

NC = 2
NS = 16
NW = NC * NS
CHUNK = 128


def _sc_mesh():
    return plsc.VectorSubcoreMesh(
        core_axis_name="c", subcore_axis_name="s", num_cores=NC, num_subcores=NS
    )


def _make_gather(ne_pad, n, d, nch):
    dp = d // 2

    @functools.partial(
        pl.kernel,
        mesh=_sc_mesh(),
        out_type=[
            jax.ShapeDtypeStruct((ne_pad, dp), jnp.int32),
            jax.ShapeDtypeStruct((ne_pad, dp), jnp.int32),
        ],
        scratch_types=[
            pltpu.VMEM((CHUNK,), jnp.int32),
            pltpu.VMEM((CHUNK,), jnp.int32),
            pltpu.VMEM((CHUNK, dp), jnp.int32),
            pltpu.VMEM((CHUNK, dp), jnp.int32),
            pltpu.SemaphoreType.DMA,
            pltpu.SemaphoreType.DMA,
        ],
        compiler_params=pltpu.CompilerParams(use_tc_tiling_on_sc=False),
    )
    def gather_k(x_hbm, src_hbm, dst_hbm, xs_out, xd_out, sidx, didx, srows, drows, s1, s2):
        cid = lax.axis_index("c")
        sid = lax.axis_index("s")
        wid = sid * NC + cid

        def body(c, _):
            base = (wid * nch + c) * CHUNK
            pltpu.sync_copy(src_hbm.at[pl.ds(base, CHUNK)], sidx)
            pltpu.sync_copy(dst_hbm.at[pl.ds(base, CHUNK)], didx)
            cp1 = pltpu.async_copy(x_hbm.at[sidx], srows, s1)
            cp2 = pltpu.async_copy(x_hbm.at[didx], drows, s2)
            cp1.wait()
            cp2.wait()
            pltpu.sync_copy(srows, xs_out.at[pl.ds(base, CHUNK)])
            pltpu.sync_copy(drows, xd_out.at[pl.ds(base, CHUNK)])
            return 0

        lax.fori_loop(0, nch, body, 0)

    return gather_k


def _make_scatter(ne_pad, n_pad, d, nsg):
    rows_per_tile = n_pad // NS
    sg = 8 * CHUNK

    @functools.partial(
        pl.kernel,
        mesh=_sc_mesh(),
        out_type=jax.ShapeDtypeStruct((NC, n_pad, d), jnp.float32),
        scratch_types=[
            pltpu.VMEM_SHARED((n_pad, d), jnp.float32),
            pltpu.VMEM((8, CHUNK), jnp.int32),
            pltpu.VMEM((2 * CHUNK, d), jnp.float32),
            pltpu.SemaphoreType.DMA,
            pltpu.SemaphoreType.DMA,
        ],
    )
    def scatter_k(m_hbm, dst_hbm, zero_hbm, acc_out, acc, didx, mrows, s_ix, s_m):
        cid = lax.axis_index("c")
        sid = lax.axis_index("s")
        wid = sid * NC + cid

        r0 = sid * rows_per_tile
        pltpu.sync_copy(zero_hbm.at[pl.ds(r0, rows_per_tile)], acc.at[pl.ds(r0, rows_per_tile)])
        plsc.subcore_barrier()

        def body(s, _):
            base = (wid * nsg + s) * sg
            ci = pltpu.async_copy(dst_hbm.at[pl.ds((wid * nsg + s) * 8, 8)], didx, s_ix)
            ci.wait()
            for j in range(4):
                cm = pltpu.async_copy(
                    m_hbm.at[pl.ds(base + j * 2 * CHUNK, 2 * CHUNK)], mrows, s_m)
                cm.wait()
                for k in range(2):
                    pltpu.sync_copy(
                        mrows.at[pl.ds(k * CHUNK, CHUNK)],
                        acc.at[didx.at[2 * j + k]], add=True)
            return 0

        lax.fori_loop(0, nsg, body, 0)
        plsc.subcore_barrier()
        pltpu.sync_copy(acc.at[pl.ds(r0, rows_per_tile)], acc_out.at[cid, pl.ds(r0, rows_per_tile)])

    return scatter_k


def _mlp_body(ne, be, xs_ref, xd_ref, e_ref,
              aw1s_ref, aw1d_ref, aw1e_ref, ab1_ref, aw2_ref, ab2_ref, aw3_ref, ab3_ref,
              mw1s_ref, mw1d_ref, mw1e_ref, mb1_ref, mw2_ref, mb2_ref, out_ref):
    def unpack(p32):
        lo = jax.lax.bitcast_convert_type(p32 << 16, jnp.float32)
        hi = jax.lax.bitcast_convert_type(p32 & jnp.int32(-65536), jnp.float32)
        return jnp.concatenate([lo, hi], axis=-1).astype(jnp.bfloat16)

    xs = unpack(xs_ref[...])
    xd = unpack(xd_ref[...])
    e = e_ref[...].astype(jnp.bfloat16)
    dot = functools.partial(jnp.dot, preferred_element_type=jnp.float32)
    t = dot(xs, aw1s_ref[...]) + dot(xd, aw1d_ref[...]) + dot(e, aw1e_ref[...]) + ab1_ref[...]
    t = jnp.maximum(t, 0.0).astype(jnp.bfloat16)
    t = jnp.maximum(dot(t, aw2_ref[...]) + ab2_ref[...], 0.0)
    a = jnp.sum(t * aw3_ref[...], axis=-1, keepdims=True) + ab3_ref[...]
    h = dot(xs, mw1s_ref[...]) + dot(xd, mw1d_ref[...]) + dot(e, mw1e_ref[...]) + mb1_ref[...]
    h = jnp.maximum(h, 0.0).astype(jnp.bfloat16)
    m = dot(h, mw2_ref[...]) + mb2_ref[...]
    msg = jax.nn.sigmoid(a) * m
    row = pl.program_id(0) * be + lax.broadcasted_iota(jnp.int32, msg.shape, 0)
    out_ref[...] = jnp.where(row < ne, msg, 0.0)


def _combine_body(x_ref, a_ref, out_ref):
    out_ref[...] = x_ref[...] + a_ref[0] + a_ref[1]


def kernel(X, E, emb_nodes, emb_edges, edge_index,
           att_W1, att_b1, att_W2, att_b2, att_W3, att_b3,
           msg_W1, msg_b1, msg_W2, msg_b2):
    n, d = X.shape
    ne = E.shape[0]
    sg = 8 * CHUNK
    nsg = -(-ne // (NW * sg))
    ne_pad = nsg * NW * sg
    pad = ne_pad - ne

    src = jnp.concatenate([edge_index[0], jnp.zeros((pad,), jnp.int32)])
    dst = jnp.concatenate([edge_index[1], jnp.zeros((pad,), jnp.int32)])
    dst2 = dst.reshape(ne_pad // CHUNK, CHUNK)
    e_pad = jnp.concatenate([E, jnp.zeros((pad, d), jnp.float32)], axis=0)

    xb = X.astype(jnp.bfloat16)
    lo16 = jax.lax.bitcast_convert_type(xb[:, : d // 2], jnp.uint16).astype(jnp.uint32)
    hi16 = jax.lax.bitcast_convert_type(xb[:, d // 2:], jnp.uint16).astype(jnp.uint32)
    xpack = jax.lax.bitcast_convert_type((hi16 << 16) | lo16, jnp.int32)

    nch = nsg * 8
    xs, xd = _make_gather(ne_pad, n, d, nch)(xpack, src, dst)

    be = 2048
    grid = ne_pad // be

    def full(shape):
        return pl.BlockSpec(shape, lambda i: tuple(0 for _ in shape))

    m_arr = pl.pallas_call(
        functools.partial(_mlp_body, ne, be),
        grid=(grid,),
        in_specs=[
            pl.BlockSpec((be, d // 2), lambda i: (i, 0)),
            pl.BlockSpec((be, d // 2), lambda i: (i, 0)),
            pl.BlockSpec((be, d), lambda i: (i, 0)),
            full((d, 96)), full((d, 96)), full((d, 96)), full((1, 96)),
            full((96, 48)), full((1, 48)), full((1, 48)), full((1, 1)),
            full((d, 256)), full((d, 256)), full((d, 256)), full((1, 256)),
            full((256, d)), full((1, d)),
        ],
        out_specs=pl.BlockSpec((be, d), lambda i: (i, 0)),
        out_shape=jax.ShapeDtypeStruct((ne_pad, d), jnp.float32),
        compiler_params=pltpu.CompilerParams(
            dimension_semantics=("arbitrary",),
        ),
    )(
        xs, xd, e_pad,
        att_W1[:d].astype(jnp.bfloat16), att_W1[d:2 * d].astype(jnp.bfloat16),
        att_W1[2 * d:].astype(jnp.bfloat16), att_b1[None, :],
        att_W2.astype(jnp.bfloat16), att_b2[None, :], att_W3.T, att_b3[None, :],
        msg_W1[:d].astype(jnp.bfloat16), msg_W1[d:2 * d].astype(jnp.bfloat16),
        msg_W1[2 * d:].astype(jnp.bfloat16), msg_b1[None, :],
        msg_W2.astype(jnp.bfloat16), msg_b2[None, :],
    )

    n_pad = NS * 8 * (-(-n // (NS * 8)))
    zeros_nd = jnp.zeros((n_pad, d), jnp.float32)
    accs = _make_scatter(ne_pad, n_pad, d, nsg)(m_arr, dst2, zeros_nd)

    bn = 2000
    x_out = pl.pallas_call(
        _combine_body,
        grid=(n // bn,),
        in_specs=[
            pl.BlockSpec((bn, d), lambda i: (i, 0)),
            pl.BlockSpec((NC, bn, d), lambda i: (0, i, 0)),
        ],
        out_specs=pl.BlockSpec((bn, d), lambda i: (i, 0)),
        out_shape=jax.ShapeDtypeStruct((n, d), jnp.float32),
    )(X, accs)

    return (x_out, E)


# speedup vs baseline: 1.5860x; 1.4614x over previous
import functools

import jax
import jax.numpy as jnp
from jax import lax
from jax.experimental import pallas as pl
from jax.experimental.pallas import tpu as pltpu
from jax.experimental.pallas import tpu_sc as plsc

NC = 2
NS = 16
NW = NC * NS
CHUNK = 128


def _sc_mesh():
    return plsc.VectorSubcoreMesh(
        core_axis_name="c", subcore_axis_name="s", num_cores=NC, num_subcores=NS
    )


def _make_gather(ne_c, n, d, tch):
    dp = d // 2

    @functools.partial(
        pl.kernel,
        mesh=_sc_mesh(),
        out_type=jax.ShapeDtypeStruct((ne_c, d), jnp.int32),
        scratch_types=[
            pltpu.VMEM((CHUNK,), jnp.int32),
            pltpu.VMEM((CHUNK,), jnp.int32),
            pltpu.VMEM((CHUNK, dp), jnp.int32),
            pltpu.VMEM((CHUNK, dp), jnp.int32),
            pltpu.SemaphoreType.DMA,
            pltpu.SemaphoreType.DMA,
        ],
        compiler_params=pltpu.CompilerParams(use_tc_tiling_on_sc=False),
    )
    def gather_k(x_hbm, src_hbm, dst_hbm, xg_out, sidx, didx, srows, drows, s1, s2):
        cid = lax.axis_index("c")
        sid = lax.axis_index("s")
        wid = sid * NC + cid
        c0 = (tch * wid) // NW
        c1 = (tch * (wid + 1)) // NW

        def body(c, _):
            base = c * CHUNK
            pltpu.sync_copy(src_hbm.at[pl.ds(base, CHUNK)], sidx)
            pltpu.sync_copy(dst_hbm.at[pl.ds(base, CHUNK)], didx)
            cp1 = pltpu.async_copy(x_hbm.at[sidx], srows, s1)
            cp2 = pltpu.async_copy(x_hbm.at[didx], drows, s2)
            cp1.wait()
            cp2.wait()
            pltpu.sync_copy(srows, xg_out.at[pl.ds(base, CHUNK), pl.ds(0, dp)])
            pltpu.sync_copy(drows, xg_out.at[pl.ds(base, CHUNK), pl.ds(dp, dp)])
            return 0

        lax.fori_loop(c0, c1, body, 0)

    return gather_k


def _make_scatter(ne_c, n_pad, d, tch):
    rows_per_tile = n_pad // NS

    @functools.partial(
        pl.kernel,
        mesh=_sc_mesh(),
        out_type=jax.ShapeDtypeStruct((NC, n_pad, d), jnp.float32),
        scratch_types=[
            pltpu.VMEM_SHARED((n_pad, d), jnp.float32),
            pltpu.VMEM((CHUNK,), jnp.int32),
            pltpu.VMEM((CHUNK, d), jnp.float32),
        ],
    )
    def scatter_k(m_hbm, dst_hbm, zero_hbm, acc_out, acc, didx, mrows):
        cid = lax.axis_index("c")
        sid = lax.axis_index("s")
        wid = sid * NC + cid
        c0 = (tch * wid) // NW
        c1 = (tch * (wid + 1)) // NW

        r0 = sid * rows_per_tile
        pltpu.sync_copy(zero_hbm.at[pl.ds(r0, rows_per_tile)], acc.at[pl.ds(r0, rows_per_tile)])
        plsc.subcore_barrier()

        def body(c, _):
            base = c * CHUNK
            pltpu.sync_copy(dst_hbm.at[pl.ds(base, CHUNK)], didx)
            pltpu.sync_copy(m_hbm.at[pl.ds(base, CHUNK)], mrows)
            pltpu.sync_copy(mrows, acc.at[didx], add=True)
            return 0

        lax.fori_loop(c0, c1, body, 0)
        plsc.subcore_barrier()
        pltpu.sync_copy(acc.at[pl.ds(r0, rows_per_tile)], acc_out.at[cid, pl.ds(r0, rows_per_tile)])

    return scatter_k


def _mlp_body(ne, be, xg_ref, e_ref,
              aw1s_ref, aw1d_ref, aw1e_ref, ab1_ref, aw2_ref, ab2_ref, aw3_ref, ab3_ref,
              mw1s_ref, mw1d_ref, mw1e_ref, mb1_ref, mw2_ref, mb2_ref, out_ref):
    dp = xg_ref.shape[1] // 2

    def unpack(p32):
        lo = jax.lax.bitcast_convert_type(p32 << 16, jnp.float32)
        hi = jax.lax.bitcast_convert_type(p32 & jnp.int32(-65536), jnp.float32)
        return jnp.concatenate([lo, hi], axis=-1).astype(jnp.bfloat16)

    xg = xg_ref[...]
    xs = unpack(xg[:, :dp])
    xd = unpack(xg[:, dp:])
    e = e_ref[...].astype(jnp.bfloat16)
    dot = functools.partial(jnp.dot, preferred_element_type=jnp.float32)
    t = dot(xs, aw1s_ref[...]) + dot(xd, aw1d_ref[...]) + dot(e, aw1e_ref[...]) + ab1_ref[...]
    t = jnp.maximum(t, 0.0).astype(jnp.bfloat16)
    t = jnp.maximum(dot(t, aw2_ref[...]) + ab2_ref[...], 0.0)
    a = jnp.sum(t * aw3_ref[...], axis=-1, keepdims=True) + ab3_ref[...]
    h = dot(xs, mw1s_ref[...]) + dot(xd, mw1d_ref[...]) + dot(e, mw1e_ref[...]) + mb1_ref[...]
    h = jnp.maximum(h, 0.0).astype(jnp.bfloat16)
    m = dot(h, mw2_ref[...]) + mb2_ref[...]
    msg = jax.nn.sigmoid(a) * m
    row = pl.program_id(0) * be + lax.broadcasted_iota(jnp.int32, msg.shape, 0)
    out_ref[...] = jnp.where(row < ne, msg, 0.0)


def _combine_body(x_ref, a_ref, out_ref):
    out_ref[...] = x_ref[...] + a_ref[0] + a_ref[1]


def kernel(X, E, emb_nodes, emb_edges, edge_index,
           att_W1, att_b1, att_W2, att_b2, att_W3, att_b3,
           msg_W1, msg_b1, msg_W2, msg_b2):
    n, d = X.shape
    ne = E.shape[0]
    tch = -(-ne // CHUNK)
    ne_c = tch * CHUNK
    pad = ne_c - ne

    if pad:
        src = jnp.concatenate([edge_index[0], jnp.zeros((pad,), jnp.int32)])
        dst = jnp.concatenate([edge_index[1], jnp.zeros((pad,), jnp.int32)])
        e_in = jnp.concatenate([E, jnp.zeros((pad, d), jnp.float32)], axis=0)
    else:
        src = edge_index[0]
        dst = edge_index[1]
        e_in = E

    xb = X.astype(jnp.bfloat16)
    lo16 = jax.lax.bitcast_convert_type(xb[:, : d // 2], jnp.uint16).astype(jnp.uint32)
    hi16 = jax.lax.bitcast_convert_type(xb[:, d // 2:], jnp.uint16).astype(jnp.uint32)
    xpack = jax.lax.bitcast_convert_type((hi16 << 16) | lo16, jnp.int32)

    xg = _make_gather(ne_c, n, d, tch)(xpack, src, dst)

    be = 2000 if ne_c % 2000 == 0 else CHUNK
    grid = ne_c // be

    def full(shape):
        return pl.BlockSpec(shape, lambda i: tuple(0 for _ in shape))

    m_arr = pl.pallas_call(
        functools.partial(_mlp_body, ne, be),
        grid=(grid,),
        in_specs=[
            pl.BlockSpec((be, d), lambda i: (i, 0)),
            pl.BlockSpec((be, d), lambda i: (i, 0)),
            full((d, 96)), full((d, 96)), full((d, 96)), full((1, 96)),
            full((96, 48)), full((1, 48)), full((1, 48)), full((1, 1)),
            full((d, 256)), full((d, 256)), full((d, 256)), full((1, 256)),
            full((256, d)), full((1, d)),
        ],
        out_specs=pl.BlockSpec((be, d), lambda i: (i, 0)),
        out_shape=jax.ShapeDtypeStruct((ne_c, d), jnp.float32),
        compiler_params=pltpu.CompilerParams(
            dimension_semantics=("arbitrary",),
        ),
    )(
        xg, e_in,
        att_W1[:d].astype(jnp.bfloat16), att_W1[d:2 * d].astype(jnp.bfloat16),
        att_W1[2 * d:].astype(jnp.bfloat16), att_b1[None, :],
        att_W2.astype(jnp.bfloat16), att_b2[None, :], att_W3.T, att_b3[None, :],
        msg_W1[:d].astype(jnp.bfloat16), msg_W1[d:2 * d].astype(jnp.bfloat16),
        msg_W1[2 * d:].astype(jnp.bfloat16), msg_b1[None, :],
        msg_W2.astype(jnp.bfloat16), msg_b2[None, :],
    )

    n_pad = NS * 8 * (-(-n // (NS * 8)))
    zeros_nd = jnp.zeros((n_pad, d), jnp.float32)
    accs = _make_scatter(ne_c, n_pad, d, tch)(m_arr, dst, zeros_nd)

    bn = 2000
    x_out = pl.pallas_call(
        _combine_body,
        grid=(n // bn,),
        in_specs=[
            pl.BlockSpec((bn, d), lambda i: (i, 0)),
            pl.BlockSpec((NC, bn, d), lambda i: (0, i, 0)),
        ],
        out_specs=pl.BlockSpec((bn, d), lambda i: (i, 0)),
        out_shape=jax.ShapeDtypeStruct((n, d), jnp.float32),
    )(X, accs)

    return (x_out, E)
